# baseline (speedup 1.0000x reference)
import jax
import jax.numpy as jnp
from jax.experimental import pallas as pl
from jax.experimental.pallas import tpu as pltpu


def _prep_body(cb_ref, w_ref, wcb_ref, cb2_ref):
    wa = jnp.abs(w_ref[0, :])
    wc = cb_ref[...] * wa[None, :]
    cb2_ref[...] = jnp.sum(wc * wc, axis=1, keepdims=True).T
    wcb_ref[...] = (-2.0 * wc).astype(jnp.bfloat16)


def _nsvq_body(x_ref, wcb_ref, cb2_ref, w_ref, rv_ref, o_ref,
               wxb_ref, sa_ref, sb_ref, acc_ref):
    j = pl.program_id(1)
    nj = pl.num_programs(1) - 1
    bk = sa_ref.shape[1]

    wa = jnp.abs(w_ref[0, :])

    @pl.when(j == 0)
    def _prep_input_tile():
        wxb_ref[...] = (x_ref[...] * wa[None, :]).astype(jnp.bfloat16)

    even = jax.lax.rem(j, 2) == 0

    @pl.when((j < nj) & even)
    def _dot_a():
        sa_ref[...] = jnp.dot(wxb_ref[...], wcb_ref[pl.ds(j * bk, bk), :].T,
                              preferred_element_type=jnp.float32)

    @pl.when((j < nj) & jnp.logical_not(even))
    def _dot_b():
        sb_ref[...] = jnp.dot(wxb_ref[...], wcb_ref[pl.ds(j * bk, bk), :].T,
                              preferred_element_type=jnp.float32)

    jm1 = jnp.maximum(j - 1, 0)

    def _consume(src_ref):
        bias = cb2_ref[0, pl.ds(jm1 * bk, bk)][None, :]
        m = jnp.min(src_ref[...] + bias, axis=1, keepdims=True)

        @pl.when(j == 1)
        def _init():
            acc_ref[...] = m

        @pl.when(j > 1)
        def _acc():
            acc_ref[...] = jnp.minimum(acc_ref[...], m)

    @pl.when((j > 0) & jnp.logical_not(even))
    def _consume_a():
        _consume(sa_ref)

    @pl.when((j > 0) & even)
    def _consume_b():
        _consume(sb_ref)

    @pl.when(j == nj)
    def _epilogue():
        eps = 1e-12
        x = x_ref[...]
        wx = x * wa[None, :]
        in2 = jnp.sum(wx * wx, axis=1, keepdims=True)
        dmin = acc_ref[...] + in2
        rv = rv_ref[...]
        nrand = jnp.sqrt(jnp.sum(rv * rv, axis=1, keepdims=True))
        nbest = jnp.sqrt(jnp.maximum(dmin, 0.0))
        scale = nbest / nrand + eps
        o_ref[...] = x + scale * rv * (1.0 / (wa[None, :] + eps))


@jax.jit
def kernel(input, codebooks, weights, random_vector):
    n, d = input.shape
    kk = codebooks.shape[0]
    bn = min(2048, n)
    bk = min(1024, kk)
    w2d = weights.reshape(1, d)

    wcb, cb2 = pl.pallas_call(
        _prep_body,
        in_specs=[
            pl.BlockSpec((kk, d), lambda: (0, 0)),
            pl.BlockSpec((1, d), lambda: (0, 0)),
        ],
        out_specs=[
            pl.BlockSpec((kk, d), lambda: (0, 0)),
            pl.BlockSpec((1, kk), lambda: (0, 0)),
        ],
        out_shape=[
            jax.ShapeDtypeStruct((kk, d), jnp.bfloat16),
            jax.ShapeDtypeStruct((1, kk), jnp.float32),
        ],
    )(codebooks, w2d)

    grid = (n // bn, kk // bk + 1)
    return pl.pallas_call(
        _nsvq_body,
        grid=grid,
        in_specs=[
            pl.BlockSpec((bn, d), lambda i, j: (i, 0)),
            pl.BlockSpec((kk, d), lambda i, j: (0, 0)),
            pl.BlockSpec((1, kk), lambda i, j: (0, 0)),
            pl.BlockSpec((1, d), lambda i, j: (0, 0)),
            pl.BlockSpec((bn, d), lambda i, j: (i, 0)),
        ],
        out_specs=pl.BlockSpec((bn, d), lambda i, j: (i, 0)),
        out_shape=jax.ShapeDtypeStruct((n, d), jnp.float32),
        scratch_shapes=[
            pltpu.VMEM((bn, d), jnp.bfloat16),
            pltpu.VMEM((bn, bk), jnp.float32),
            pltpu.VMEM((bn, bk), jnp.float32),
            pltpu.VMEM((bn, 1), jnp.float32),
        ],
        compiler_params=pltpu.CompilerParams(
            dimension_semantics=("parallel", "arbitrary"),
        ),
    )(input, wcb, cb2, w2d, random_vector)


# speedup vs baseline: 1.4230x; 1.4230x over previous
import jax
import jax.numpy as jnp
from jax.experimental import pallas as pl
from jax.experimental.pallas import tpu as pltpu


def _prep_body(cb_ref, w_ref, wcb_ref, cb2_ref):
    wa = jnp.abs(w_ref[0, :])
    wc = cb_ref[...] * wa[None, :]
    cb2_ref[...] = jnp.sum(wc * wc, axis=1, keepdims=True).T
    wcb_ref[...] = (-2.0 * wc).astype(jnp.bfloat16)


def _nsvq_body(x_ref, wcb_ref, cb2_ref, w_ref, rv_ref, o_ref,
               wxb_ref, acc_ref):
    j = pl.program_id(1)
    nj = pl.num_programs(1)
    bk = wcb_ref.shape[0] // nj

    wa = jnp.abs(w_ref[0, :])

    @pl.when(j == 0)
    def _prep_input_tile():
        wxb_ref[...] = (x_ref[...] * wa[None, :]).astype(jnp.bfloat16)

    sc = jnp.dot(wxb_ref[...], wcb_ref[pl.ds(j * bk, bk), :].T,
                 preferred_element_type=jnp.float32)
    m = jnp.min(sc + cb2_ref[0, pl.ds(j * bk, bk)][None, :],
                axis=1, keepdims=True)

    @pl.when(j == 0)
    def _init():
        acc_ref[...] = m

    @pl.when(j > 0)
    def _acc():
        acc_ref[...] = jnp.minimum(acc_ref[...], m)

    @pl.when(j == nj - 1)
    def _epilogue():
        eps = 1e-12
        x = x_ref[...]
        wx = x * wa[None, :]
        in2 = jnp.sum(wx * wx, axis=1, keepdims=True)
        dmin = acc_ref[...] + in2
        rv = rv_ref[...]
        nrand = jnp.sqrt(jnp.sum(rv * rv, axis=1, keepdims=True))
        nbest = jnp.sqrt(jnp.maximum(dmin, 0.0))
        scale = nbest / nrand + eps
        o_ref[...] = x + scale * rv * (1.0 / (wa[None, :] + eps))


@jax.jit
def kernel(input, codebooks, weights, random_vector):
    n, d = input.shape
    kk = codebooks.shape[0]
    bn = min(4096, n)
    bk = min(1024, kk)
    w2d = weights.reshape(1, d)

    wcb, cb2 = pl.pallas_call(
        _prep_body,
        in_specs=[
            pl.BlockSpec((kk, d), lambda: (0, 0)),
            pl.BlockSpec((1, d), lambda: (0, 0)),
        ],
        out_specs=[
            pl.BlockSpec((kk, d), lambda: (0, 0)),
            pl.BlockSpec((1, kk), lambda: (0, 0)),
        ],
        out_shape=[
            jax.ShapeDtypeStruct((kk, d), jnp.bfloat16),
            jax.ShapeDtypeStruct((1, kk), jnp.float32),
        ],
    )(codebooks, w2d)

    grid = (n // bn, kk // bk)
    return pl.pallas_call(
        _nsvq_body,
        grid=grid,
        in_specs=[
            pl.BlockSpec((bn, d), lambda i, j: (i, 0)),
            pl.BlockSpec((kk, d), lambda i, j: (0, 0)),
            pl.BlockSpec((1, kk), lambda i, j: (0, 0)),
            pl.BlockSpec((1, d), lambda i, j: (0, 0)),
            pl.BlockSpec((bn, d), lambda i, j: (i, 0)),
        ],
        out_specs=pl.BlockSpec((bn, d), lambda i, j: (i, 0)),
        out_shape=jax.ShapeDtypeStruct((n, d), jnp.float32),
        scratch_shapes=[
            pltpu.VMEM((bn, d), jnp.bfloat16),
            pltpu.VMEM((bn, 1), jnp.float32),
        ],
        compiler_params=pltpu.CompilerParams(
            dimension_semantics=("parallel", "arbitrary"),
        ),
    )(input, wcb, cb2, w2d, random_vector)
